# baseline (speedup 1.0000x reference)
import functools

import jax
import jax.numpy as jnp
from jax import lax
from jax.experimental import pallas as pl
from jax.experimental.pallas import tpu as pltpu
from jax.experimental.pallas import tpu_sc as plsc

_B, _S, _D = 4, 128, 128
_NB, _E = 32, 64
_C = 3 * _D + _E
_R = 16
_NBUF = 8
_LUT_N = 2048
_LANES = 16


def _bucket_lut(num_identity_buckets=4, num_total_buckets=_NB):
    d = jnp.arange(_LUT_N, dtype=jnp.int32)
    df = d.astype(jnp.float32)
    logspace_idx = jnp.floor(
        jnp.log(jnp.maximum(df, 1.0)) / jnp.log(2.0)).astype(jnp.int32) + 3
    combined = jnp.where(d <= num_identity_buckets, d, logspace_idx)
    return jnp.clip(combined, 0, num_total_buckets - 1)


def _sc_buckets(begin_flat, end_flat, lut):
    n_rows = _B * _S
    info = plsc.get_sparse_core_info()
    nw = info.num_cores * info.num_subcores
    rows_per_w = n_rows // nw

    mesh = plsc.VectorSubcoreMesh(core_axis_name="c", subcore_axis_name="s")

    @functools.partial(
        pl.kernel,
        mesh=mesh,
        out_type=jax.ShapeDtypeStruct((n_rows, _S), jnp.int32),
        compiler_params=pltpu.CompilerParams(needs_layout_passes=False),
        scratch_types=[
            pltpu.VMEM((_S,), jnp.int32),
            pltpu.VMEM((_S,), jnp.int32),
            pltpu.VMEM((_LUT_N,), jnp.int32),
            pltpu.VMEM((rows_per_w, _S), jnp.int32),
        ],
    )
    def k(begin_hbm, end_hbm, lut_hbm, out_hbm, beg_v, end_v, lut_v, rows_v):
        wid = lax.axis_index("s") * info.num_cores + lax.axis_index("c")
        row0 = wid * rows_per_w
        b = row0 // _S
        pltpu.sync_copy(begin_hbm.at[pl.ds(b * _S, _S)], beg_v)
        pltpu.sync_copy(end_hbm.at[pl.ds(b * _S, _S)], end_v)
        pltpu.sync_copy(lut_hbm, lut_v)
        i_base = row0 % _S
        for kk in range(rows_per_w):
            idx_i = jnp.full((_LANES,), i_base + kk, jnp.int32)
            bi = plsc.load_gather(beg_v, [idx_i])
            ei = plsc.load_gather(end_v, [idx_i])
            for jc in range(_S // _LANES):
                bj = beg_v[pl.ds(jc * _LANES, _LANES)]
                ej = end_v[pl.ds(jc * _LANES, _LANES)]
                dd = jnp.maximum(bi - ej, 0) + jnp.maximum(bj - ei, 0)
                dd = jnp.minimum(dd, _LUT_N - 1)
                rows_v[kk, pl.ds(jc * _LANES, _LANES)] = plsc.load_gather(
                    lut_v, [dd])
        pltpu.sync_copy(rows_v, out_hbm.at[pl.ds(row0, rows_per_w)])

    return k(begin_flat, end_flat, lut)


_NCHUNK = 2
_CROWS = _B * _S // _NCHUNK
_CSTEP = _CROWS // _R


def _make_tc_body(row0):
    def body(sv_ref, bk_ref, emb_ref, out_ref, buf, sems):
        step = pl.program_id(0)
        i0 = (row0 + step * _R) % _S
        slot = lax.rem(step, _NBUF)

        @pl.when(step >= _NBUF)
        def _():
            old = step - _NBUF
            pltpu.make_async_copy(
                buf.at[slot], out_ref.at[pl.ds(old * _R, _R)], sems.at[slot]
            ).wait()

        sv = sv_ref[0]
        rows = sv_ref[0, pl.ds(i0, _R), :]
        left = jnp.broadcast_to(rows[:, None, :], (_R, _S, _D))
        right = jnp.broadcast_to(sv[None, :, :], (_R, _S, _D))
        bk = bk_ref[...]
        iot = lax.broadcasted_iota(jnp.int32, (_R, _S, _NB), 2)
        onehot = (bk[:, :, None] == iot).astype(jnp.float32)
        demb = lax.dot_general(
            onehot.reshape(_R * _S, _NB), emb_ref[...],
            (((1,), (0,)), ((), ())),
            preferred_element_type=jnp.float32).reshape(_R, _S, _E)
        buf[slot, :, :, 0:_D] = left
        buf[slot, :, :, _D:2 * _D] = right
        buf[slot, :, :, 2 * _D:3 * _D] = left * right
        buf[slot, :, :, 3 * _D:] = demb
        pltpu.make_async_copy(
            buf.at[slot], out_ref.at[pl.ds(step * _R, _R)], sems.at[slot]
        ).start()

        @pl.when(step == _CSTEP - 1)
        def _():
            for k in range(_NBUF):
                st = _CSTEP - _NBUF + k
                sl = st % _NBUF
                pltpu.make_async_copy(
                    buf.at[sl], out_ref.at[pl.ds(st * _R, _R)], sems.at[sl]
                ).wait()

    return body


def _tc_chunk(span_vecs, buckets, dist_emb, row0):
    return pl.pallas_call(
        _make_tc_body(row0),
        grid=(_CSTEP,),
        in_specs=[
            pl.BlockSpec((1, _S, _D),
                         lambda s, row0=row0: ((row0 + s * _R) // _S, 0, 0)),
            pl.BlockSpec((_R, _S), lambda s, row0=row0: (row0 // _R + s, 0)),
            pl.BlockSpec((_NB, _E), lambda s: (0, 0)),
        ],
        out_specs=pl.BlockSpec(memory_space=pl.ANY),
        out_shape=jax.ShapeDtypeStruct((_CROWS, _S, _C), jnp.float32),
        scratch_shapes=[
            pltpu.VMEM((_NBUF, _R, _S, _C), jnp.float32),
            pltpu.SemaphoreType.DMA((_NBUF,)),
        ],
    )(span_vecs, buckets, dist_emb)


def kernel(span_vecs, span_begin, span_end, dist_emb):
    B, S, _ = span_vecs.shape
    lut = _bucket_lut(num_total_buckets=dist_emb.shape[0])
    bk = _sc_buckets(span_begin.reshape(B * S), span_end.reshape(B * S), lut)
    chunks = [_tc_chunk(span_vecs, bk, dist_emb, c * _CROWS)
              for c in range(_NCHUNK)]
    out = jnp.concatenate(chunks, axis=0) if _NCHUNK > 1 else chunks[0]
    return out.reshape(_B, _S, _S, _C)


# speedup vs baseline: 1.5416x; 1.5416x over previous
import functools

import jax
import jax.numpy as jnp
from jax import lax
from jax.experimental import pallas as pl
from jax.experimental.pallas import tpu as pltpu
from jax.experimental.pallas import tpu_sc as plsc

_B, _S, _D = 4, 128, 128
_NB, _E = 32, 64
_C = 3 * _D + _E
_R = 16
_NBUF = 8
_LUT_N = 2048
_LANES = 16


def _bucket_lut(num_identity_buckets=4, num_total_buckets=_NB):
    d = jnp.arange(_LUT_N, dtype=jnp.int32)
    df = d.astype(jnp.float32)
    logspace_idx = jnp.floor(
        jnp.log(jnp.maximum(df, 1.0)) / jnp.log(2.0)).astype(jnp.int32) + 3
    combined = jnp.where(d <= num_identity_buckets, d, logspace_idx)
    return jnp.clip(combined, 0, num_total_buckets - 1)


def _sc_buckets(begin_flat, end_flat, lut):
    n_rows = _B * _S
    info = plsc.get_sparse_core_info()
    nw = info.num_cores * info.num_subcores
    rows_per_w = n_rows // nw

    mesh = plsc.VectorSubcoreMesh(core_axis_name="c", subcore_axis_name="s")

    @functools.partial(
        pl.kernel,
        mesh=mesh,
        out_type=jax.ShapeDtypeStruct((n_rows, _S), jnp.int32),
        compiler_params=pltpu.CompilerParams(needs_layout_passes=False),
        scratch_types=[
            pltpu.VMEM((_S,), jnp.int32),
            pltpu.VMEM((_S,), jnp.int32),
            pltpu.VMEM((_LUT_N,), jnp.int32),
            pltpu.VMEM((rows_per_w, _S), jnp.int32),
        ],
    )
    def k(begin_hbm, end_hbm, lut_hbm, out_hbm, beg_v, end_v, lut_v, rows_v):
        wid = lax.axis_index("s") * info.num_cores + lax.axis_index("c")
        row0 = wid * rows_per_w
        b = row0 // _S
        pltpu.sync_copy(begin_hbm.at[pl.ds(b * _S, _S)], beg_v)
        pltpu.sync_copy(end_hbm.at[pl.ds(b * _S, _S)], end_v)
        pltpu.sync_copy(lut_hbm, lut_v)
        i_base = row0 % _S
        for kk in range(rows_per_w):
            idx_i = jnp.full((_LANES,), i_base + kk, jnp.int32)
            bi = plsc.load_gather(beg_v, [idx_i])
            ei = plsc.load_gather(end_v, [idx_i])
            for jc in range(_S // _LANES):
                bj = beg_v[pl.ds(jc * _LANES, _LANES)]
                ej = end_v[pl.ds(jc * _LANES, _LANES)]
                dd = jnp.maximum(bi - ej, 0) + jnp.maximum(bj - ei, 0)
                dd = jnp.minimum(dd, _LUT_N - 1)
                rows_v[kk, pl.ds(jc * _LANES, _LANES)] = plsc.load_gather(
                    lut_v, [dd])
        pltpu.sync_copy(rows_v, out_hbm.at[pl.ds(row0, rows_per_w)])

    return k(begin_flat, end_flat, lut)


_NCHUNK = 1
_CROWS = _B * _S // _NCHUNK
_CSTEP = _CROWS // _R


def _make_tc_body(row0):
    def body(sv_ref, bk_ref, emb_ref, out_ref, buf, sems):
        step = pl.program_id(0)
        i0 = (row0 + step * _R) % _S
        slot = lax.rem(step, _NBUF)

        @pl.when(step >= _NBUF)
        def _():
            old = step - _NBUF
            pltpu.make_async_copy(
                buf.at[slot], out_ref.at[pl.ds(old * _R, _R)], sems.at[slot]
            ).wait()

        sv = sv_ref[0]
        rows = sv_ref[0, pl.ds(i0, _R), :]
        left = jnp.broadcast_to(rows[:, None, :], (_R, _S, _D))
        right = jnp.broadcast_to(sv[None, :, :], (_R, _S, _D))
        bk = bk_ref[...]
        iot = lax.broadcasted_iota(jnp.int32, (_R, _S, _NB), 2)
        onehot = (bk[:, :, None] == iot).astype(jnp.float32)
        demb = lax.dot_general(
            onehot.reshape(_R * _S, _NB), emb_ref[...],
            (((1,), (0,)), ((), ())),
            preferred_element_type=jnp.float32).reshape(_R, _S, _E)
        buf[slot, :, :, 0:_D] = left
        buf[slot, :, :, _D:2 * _D] = right
        buf[slot, :, :, 2 * _D:3 * _D] = left * right
        buf[slot, :, :, 3 * _D:] = demb
        pltpu.make_async_copy(
            buf.at[slot], out_ref.at[pl.ds(step * _R, _R)], sems.at[slot]
        ).start()

        @pl.when(step == _CSTEP - 1)
        def _():
            for k in range(_NBUF):
                st = _CSTEP - _NBUF + k
                sl = st % _NBUF
                pltpu.make_async_copy(
                    buf.at[sl], out_ref.at[pl.ds(st * _R, _R)], sems.at[sl]
                ).wait()

    return body


def _tc_chunk(span_vecs, buckets, dist_emb, row0):
    return pl.pallas_call(
        _make_tc_body(row0),
        grid=(_CSTEP,),
        in_specs=[
            pl.BlockSpec((1, _S, _D),
                         lambda s, row0=row0: ((row0 + s * _R) // _S, 0, 0)),
            pl.BlockSpec((_R, _S), lambda s, row0=row0: (row0 // _R + s, 0)),
            pl.BlockSpec((_NB, _E), lambda s: (0, 0)),
        ],
        out_specs=pl.BlockSpec(memory_space=pl.ANY),
        out_shape=jax.ShapeDtypeStruct((_CROWS, _S, _C), jnp.float32),
        scratch_shapes=[
            pltpu.VMEM((_NBUF, _R, _S, _C), jnp.float32),
            pltpu.SemaphoreType.DMA((_NBUF,)),
        ],
    )(span_vecs, buckets, dist_emb)


def kernel(span_vecs, span_begin, span_end, dist_emb):
    B, S, _ = span_vecs.shape
    lut = _bucket_lut(num_total_buckets=dist_emb.shape[0])
    bk = _sc_buckets(span_begin.reshape(B * S), span_end.reshape(B * S), lut)
    chunks = [_tc_chunk(span_vecs, bk, dist_emb, c * _CROWS)
              for c in range(_NCHUNK)]
    out = jnp.concatenate(chunks, axis=0) if _NCHUNK > 1 else chunks[0]
    return out.reshape(_B, _S, _S, _C)
